# baseline (speedup 1.0000x reference)
import functools

import jax
import jax.numpy as jnp
from jax.experimental import pallas as pl
from jax.experimental.pallas import tpu as pltpu

_B = 64
_F = 4 * 64 * 64
_BB = 8


def _fma_body(c1_ref, c2_ref, x_ref, n_ref, o_ref):
    o_ref[...] = c1_ref[...] * x_ref[...] + c2_ref[...] * n_ref[...]


@jax.jit
def _tc_fma(c1, c2, x2, n2):
    grid = (_B // _BB,)
    return pl.pallas_call(
        _fma_body,
        grid=grid,
        in_specs=[
            pl.BlockSpec((_BB, 1), lambda i: (i, 0)),
            pl.BlockSpec((_BB, 1), lambda i: (i, 0)),
            pl.BlockSpec((_BB, _F), lambda i: (i, 0)),
            pl.BlockSpec((_BB, _F), lambda i: (i, 0)),
        ],
        out_specs=pl.BlockSpec((_BB, _F), lambda i: (i, 0)),
        out_shape=jax.ShapeDtypeStruct((_B, _F), jnp.float32),
    )(c1, c2, x2, n2)


def kernel(x_start, noise, sqrt_alphas_cumprod, sqrt_one_minus_alphas_cumprod, t):
    c1 = jnp.take(sqrt_alphas_cumprod, t, axis=0).reshape(_B, 1)
    c2 = jnp.take(sqrt_one_minus_alphas_cumprod, t, axis=0).reshape(_B, 1)
    x2 = x_start.reshape(_B, _F)
    n2 = noise.reshape(_B, _F)
    out = _tc_fma(c1, c2, x2, n2)
    return out.reshape(x_start.shape)


# speedup vs baseline: 2.1485x; 2.1485x over previous
import functools

import jax
import jax.numpy as jnp
from jax.experimental import pallas as pl
from jax.experimental.pallas import tpu as pltpu

_B = 64
_BB = 8


def _fma_body(c1_ref, c2_ref, x_ref, n_ref, o_ref):
    o_ref[...] = c1_ref[...] * x_ref[...] + c2_ref[...] * n_ref[...]


@jax.jit
def _tc_fma(c1, c2, x, n):
    grid = (_B // _BB,)
    blk = (_BB,) + x.shape[1:]
    cblk = (_BB, 1, 1, 1)
    return pl.pallas_call(
        _fma_body,
        grid=grid,
        in_specs=[
            pl.BlockSpec(cblk, lambda i: (i, 0, 0, 0)),
            pl.BlockSpec(cblk, lambda i: (i, 0, 0, 0)),
            pl.BlockSpec(blk, lambda i: (i, 0, 0, 0)),
            pl.BlockSpec(blk, lambda i: (i, 0, 0, 0)),
        ],
        out_specs=pl.BlockSpec(blk, lambda i: (i, 0, 0, 0)),
        out_shape=jax.ShapeDtypeStruct(x.shape, jnp.float32),
    )(c1, c2, x, n)


def kernel(x_start, noise, sqrt_alphas_cumprod, sqrt_one_minus_alphas_cumprod, t):
    c1 = jnp.take(sqrt_alphas_cumprod, t, axis=0).reshape(_B, 1, 1, 1)
    c2 = jnp.take(sqrt_one_minus_alphas_cumprod, t, axis=0).reshape(_B, 1, 1, 1)
    return _tc_fma(c1, c2, x_start, noise)


# speedup vs baseline: 2.8952x; 1.3475x over previous
import functools

import jax
import jax.numpy as jnp
from jax.experimental import pallas as pl
from jax.experimental.pallas import tpu as pltpu

_B = 64
_BB = 8


def _fused_body(t_ref, sac_ref, somac_ref, x_ref, n_ref, o_ref):
    i = pl.program_id(0)
    for r in range(_BB):
        ti = t_ref[i * _BB + r]
        c1 = sac_ref[ti]
        c2 = somac_ref[ti]
        o_ref[r] = c1 * x_ref[r] + c2 * n_ref[r]


@jax.jit
def _tc_fused(t, sac, somac, x, n):
    blk = (_BB,) + x.shape[1:]
    imap = lambda i, *_: (i, 0, 0, 0)
    grid_spec = pltpu.PrefetchScalarGridSpec(
        num_scalar_prefetch=3,
        grid=(_B // _BB,),
        in_specs=[
            pl.BlockSpec(blk, imap),
            pl.BlockSpec(blk, imap),
        ],
        out_specs=pl.BlockSpec(blk, imap),
    )
    return pl.pallas_call(
        _fused_body,
        grid_spec=grid_spec,
        out_shape=jax.ShapeDtypeStruct(x.shape, jnp.float32),
    )(t, sac, somac, x, n)


def kernel(x_start, noise, sqrt_alphas_cumprod, sqrt_one_minus_alphas_cumprod, t):
    return _tc_fused(t, sqrt_alphas_cumprod, sqrt_one_minus_alphas_cumprod,
                     x_start, noise)
